# baseline (device time: 390501 ns/iter reference)
import jax
import jax.numpy as jnp
from jax import lax
from jax.experimental import pallas as pl
from jax.experimental.pallas import tpu as pltpu

N_DEV = 4
N_HOPS = N_DEV - 1
NPAIR = 4


def kernel(x, w_mat, scale_x, scale_w):
    m_total, k = x.shape
    _, n = w_mat.shape
    mb = m_total // N_DEV
    wn = n // (2 * NPAIR)

    x16 = x.astype(jnp.bfloat16)
    w16 = w_mat.astype(jnp.bfloat16)

    def body(x_ref, w_ref, sx_ref, sw_ref, out_ref,
             wbuf, commr, comml, ystage,
             sendr, recvr, sendl, recvl, wsems, outsem):
        d = lax.axis_index("i")
        nl = lax.rem(d + N_DEV - 1, N_DEV)
        nr = lax.rem(d + 1, N_DEV)
        scale = sx_ref[0] * sw_ref[0]

        def part(c, p, half):
            blk = x_ref[pl.ds(c * mb, mb), :]
            wblk = wbuf[p % 2, :, half * wn:(half + 1) * wn]
            return jnp.dot(blk, wblk, preferred_element_type=jnp.float32)

        def crecv_r(h):
            return lax.rem(d + 2 * N_DEV - 2 - h, N_DEV)

        def crecv_l(h):
            return lax.rem(d + 2 + h, N_DEV)

        wdmas = {p: pltpu.make_async_copy(
            w_ref.at[:, pl.ds(p * 2 * wn, 2 * wn)],
            wbuf.at[p % 2], wsems.at[p % 2]) for p in range(NPAIR)}

        sends = {}
        send_waited = set()

        def issue(ring, p, h):
            comm = commr if ring == "R" else comml
            ssem = sendr if ring == "R" else sendl
            rsem = recvr if ring == "R" else recvl
            tgt = nr if ring == "R" else nl
            b = p % 2
            src_kind = 2 if h == 0 else h - 1
            rdma = pltpu.make_async_remote_copy(
                src_ref=comm.at[src_kind, b],
                dst_ref=comm.at[h, b],
                send_sem=ssem.at[p, h],
                recv_sem=rsem.at[p, h],
                device_id=(tgt,),
                device_id_type=pl.DeviceIdType.MESH,
            )
            rdma.start()
            sends[(ring, p, h)] = rdma

        def drain(ring, p, h):
            if p >= 0 and (ring, p, h) not in send_waited:
                sends[(ring, p, h)].wait_send()
                send_waited.add((ring, p, h))

        wdmas[0].start()
        wdmas[1].start()
        wdmas[0].wait()
        commr[2, 0] = part(nl, 0, 0).astype(jnp.bfloat16)
        comml[2, 0] = part(nr, 0, 1).astype(jnp.bfloat16)

        barrier = pltpu.get_barrier_semaphore()
        for nbr in (nl, nr):
            pl.semaphore_signal(barrier, inc=1, device_id=(nbr,),
                                device_id_type=pl.DeviceIdType.MESH)
        pl.semaphore_wait(barrier, 2)

        issue("R", 0, 0)
        issue("L", 0, 0)

        outcp = {}
        for p in range(NPAIR):
            b, nb = p % 2, (p + 1) % 2

            if p + 1 < NPAIR:
                wdmas[p + 1].wait()
                if p >= 1:
                    drain("R", p - 1, 0)
                    drain("L", p - 1, 0)
                commr[2, nb] = part(nl, p + 1, 0).astype(jnp.bfloat16)
                comml[2, nb] = part(nr, p + 1, 1).astype(jnp.bfloat16)
                issue("R", p + 1, 0)
                issue("L", p + 1, 0)
            pr0 = part(crecv_r(0), p, 0)
            pl0 = part(crecv_l(0), p, 1)

            sends[("R", p, 0)].wait_recv()
            if p >= 2:
                drain("R", p - 2, 1)
            commr[0, b] = (commr[0, b].astype(jnp.float32) + pr0
                           ).astype(jnp.bfloat16)
            issue("R", p, 1)
            sends[("L", p, 0)].wait_recv()
            if p >= 2:
                drain("L", p - 2, 1)
            comml[0, b] = (comml[0, b].astype(jnp.float32) + pl0
                           ).astype(jnp.bfloat16)
            issue("L", p, 1)

            pr1 = part(crecv_r(1), p, 0)
            pl1 = part(crecv_l(1), p, 1)

            sends[("R", p, 1)].wait_recv()
            if p >= 2:
                drain("R", p - 2, 2)
            commr[1, b] = (commr[1, b].astype(jnp.float32) + pr1
                           ).astype(jnp.bfloat16)
            issue("R", p, 2)
            sends[("L", p, 1)].wait_recv()
            if p >= 2:
                drain("L", p - 2, 2)
            comml[1, b] = (comml[1, b].astype(jnp.float32) + pl1
                           ).astype(jnp.bfloat16)
            issue("L", p, 2)

            pr2 = part(d, p, 0)
            if p >= 1:
                outcp[p - 1].wait()

            sends[("R", p, 2)].wait_recv()
            ystage[...] = (commr[2, b].astype(jnp.float32) + pr2) * scale
            ocp_r = pltpu.make_async_copy(
                ystage, out_ref.at[:, pl.ds(2 * p * wn, wn)], outsem)
            ocp_r.start()
            pl2 = part(d, p, 1)
            if p + 2 < NPAIR:
                wdmas[p + 2].start()
            sends[("L", p, 2)].wait_recv()
            ocp_r.wait()
            ystage[...] = (comml[2, b].astype(jnp.float32) + pl2) * scale
            ocp = pltpu.make_async_copy(
                ystage, out_ref.at[:, pl.ds((2 * p + 1) * wn, wn)], outsem)
            ocp.start()
            outcp[p] = ocp

        outcp[NPAIR - 1].wait()
        for key in list(sends):
            drain(*key)

    out_shape = jax.ShapeDtypeStruct((mb, n), jnp.float32)
    return pl.pallas_call(
        body,
        out_shape=out_shape,
        in_specs=[
            pl.BlockSpec(memory_space=pltpu.VMEM),
            pl.BlockSpec(memory_space=pl.ANY),
            pl.BlockSpec(memory_space=pltpu.SMEM),
            pl.BlockSpec(memory_space=pltpu.SMEM),
        ],
        out_specs=pl.BlockSpec(memory_space=pl.ANY),
        scratch_shapes=[
            pltpu.VMEM((2, k, 2 * wn), jnp.bfloat16),
            pltpu.VMEM((N_HOPS, 2, mb, wn), jnp.bfloat16),
            pltpu.VMEM((N_HOPS, 2, mb, wn), jnp.bfloat16),
            pltpu.VMEM((mb, wn), jnp.float32),
            pltpu.SemaphoreType.DMA((NPAIR, N_HOPS)),
            pltpu.SemaphoreType.DMA((NPAIR, N_HOPS)),
            pltpu.SemaphoreType.DMA((NPAIR, N_HOPS)),
            pltpu.SemaphoreType.DMA((NPAIR, N_HOPS)),
            pltpu.SemaphoreType.DMA((2,)),
            pltpu.SemaphoreType.DMA,
        ],
        compiler_params=pltpu.CompilerParams(
            collective_id=0, vmem_limit_bytes=58 * 1024 * 1024),
    )(x16, w16, scale_x, scale_w)
